# baseline (speedup 1.0000x reference)
import jax
import jax.numpy as jnp
from jax import lax
from jax.experimental import pallas as pl
from jax.experimental.pallas import tpu as pltpu
from jax.experimental.pallas import tpu_sc as plsc

B = 100000
D = 128
NC = 2
NS = 16
NW = NC * NS
CHUNK = 256
GPC = CHUNK // 128
NCHUNK = (B + CHUNK - 1) // CHUNK
LAST_BASE = B - CHUNK
MAX_LOC = (NCHUNK + NW - 1) // NW
NBUF = 3


def _body(z_hbm, table_hbm, out_hbm, i0, i1, i2, r0, r1, r2,
          g0, g1, g2, w0, w1, w2):
    wid = lax.axis_index("s") * NC + lax.axis_index("c")
    nloc = (NCHUNK - wid + NW - 1) // NW
    idxs = (i0, i1, i2)
    rows = (r0, r1, r2)
    gsems = (g0, g1, g2)
    wsems = (w0, w1, w2)

    def base_of(i):
        cid = wid + i * NW
        return pl.multiple_of(lax.min(cid * CHUNK, LAST_BASE), 8)

    def fire_gather(i, b):
        base = base_of(i)
        pltpu.sync_copy(z_hbm.at[pl.ds(base, CHUNK)], idxs[b])
        for j in range(GPC):
            pltpu.async_copy(
                table_hbm.at[idxs[b].at[pl.ds(j * 128, 128)]],
                rows[b].at[pl.ds(j * 128, 128)],
                gsems[b],
            )

    def wait_gather(b):
        for j in range(GPC):
            pltpu.make_async_copy(
                table_hbm.at[idxs[b].at[pl.ds(j * 128, 128)]],
                rows[b].at[pl.ds(j * 128, 128)],
                gsems[b],
            ).wait()

    def fire_write(i, b):
        pltpu.async_copy(rows[b], out_hbm.at[pl.ds(base_of(i), CHUNK)], wsems[b])

    def wait_write(i, b):
        pltpu.make_async_copy(
            rows[b], out_hbm.at[pl.ds(base_of(i), CHUNK)], wsems[b]
        ).wait()

    fire_gather(0, 0)
    fire_gather(1, 1)

    def step(i, b):
        @pl.when(i < nloc)
        def _():
            wait_gather(b)
            fire_write(i, b)

        @pl.when(i + 2 < nloc)
        def _():
            b2 = (b + 2) % NBUF
            @pl.when(i >= 1)
            def _():
                wait_write(i - 1, b2)

            fire_gather(i + 2, b2)

    def loop_body(k, carry):
        step(3 * k, 0)
        step(3 * k + 1, 1)
        step(3 * k + 2, 2)
        return carry

    lax.fori_loop(0, (MAX_LOC + NBUF - 1) // NBUF, loop_body, 0)

    @pl.when(nloc == 12)
    def _():
        wait_write(9, 0)
        wait_write(10, 1)
        wait_write(11, 2)

    @pl.when(nloc == 13)
    def _():
        wait_write(10, 1)
        wait_write(11, 2)
        wait_write(12, 0)


@jax.jit
def kernel(z, table):
    z = z.astype(jnp.int32)
    mesh = plsc.VectorSubcoreMesh(core_axis_name="c", subcore_axis_name="s")
    f = pl.kernel(
        _body,
        out_type=jax.ShapeDtypeStruct((B, D), jnp.float32),
        mesh=mesh,
        scratch_types=[
            pltpu.VMEM((CHUNK,), jnp.int32),
            pltpu.VMEM((CHUNK,), jnp.int32),
            pltpu.VMEM((CHUNK,), jnp.int32),
            pltpu.VMEM((CHUNK, D), jnp.float32),
            pltpu.VMEM((CHUNK, D), jnp.float32),
            pltpu.VMEM((CHUNK, D), jnp.float32),
            pltpu.SemaphoreType.DMA,
            pltpu.SemaphoreType.DMA,
            pltpu.SemaphoreType.DMA,
            pltpu.SemaphoreType.DMA,
            pltpu.SemaphoreType.DMA,
            pltpu.SemaphoreType.DMA,
        ],
    )
    return f(z, table)
